# Initial kernel scaffold; baseline (speedup 1.0000x reference)
import jax
import jax.numpy as jnp
from jax.experimental import pallas as pl


def kernel(x, router_w, router_b, up_w, up_b, down_w, down_b):
    raise NotImplementedError("write your pallas kernel here")


# speedup vs baseline: 3.9002x; 3.9002x over previous
import functools

import jax
import jax.numpy as jnp
from jax import lax
from jax.experimental import pallas as pl
from jax.experimental.pallas import tpu as pltpu
from jax.experimental.pallas import tpu_sc as plsc

T = 2048
D = 768
E = 8
H = 3072
HU = 6144
BLK = 256
N_PAD = 6144
N_TILES = N_PAD // BLK
NW = 32
TPW = T // NW
LANES = 16


def _router_body(x_ref, rw_ref, rb_ref, meta_ref, te_ref, pw0_ref, pw1_ref):
    x = x_ref[...]
    rw = rw_ref[...]
    logits = lax.dot_general(x, rw, (((1,), (1,)), ((), ())),
                             preferred_element_type=jnp.float32)
    logits = logits + rb_ref[...]

    eidx = lax.broadcasted_iota(jnp.int32, (T, E), 1)
    m1 = jnp.max(logits, axis=1, keepdims=True)
    is1 = logits == m1
    idx1 = jnp.min(jnp.where(is1, eidx, E), axis=1, keepdims=True)
    masked = jnp.where(eidx == idx1, -jnp.inf, logits)
    m2 = jnp.max(masked, axis=1, keepdims=True)
    idx2 = jnp.min(jnp.where(masked == m2, eidx, E), axis=1, keepdims=True)

    eb = jnp.exp(m2 - m1)
    p1 = 1.0 / (1.0 + eb)
    p2 = eb * p1

    oh1 = (eidx == idx1).astype(jnp.float32)
    oh2 = (eidx == idx2).astype(jnp.float32)
    m_sel = oh1 + oh2

    tr = lax.broadcasted_iota(jnp.int32, (T, T), 0)
    tc = lax.broadcasted_iota(jnp.int32, (T, T), 1)
    tri = (tr >= tc).astype(jnp.float32)
    cum = lax.dot_general(tri, m_sel, (((1,), (0,)), ((), ())),
                          preferred_element_type=jnp.float32)
    excl = cum - m_sel

    counts_row = cum[T - 1:T, :]
    ones_t = jnp.full((T, 1), 1.0, dtype=jnp.float32)
    counts_col = lax.dot_general(m_sel, ones_t, (((0,), (0,)), ((), ())),
                                 preferred_element_type=jnp.float32)

    inv_blk = 1.0 / BLK
    pc_row = jnp.floor((counts_row + (BLK - 1)) * inv_blk) * BLK
    pc_col = jnp.floor((counts_col + (BLK - 1)) * inv_blk) * BLK

    er = lax.broadcasted_iota(jnp.int32, (E, E), 0)
    ec = lax.broadcasted_iota(jnp.int32, (E, E), 1)
    le = (er <= ec).astype(jnp.float32)
    off_incl_row = lax.dot_general(pc_row, le, (((1,), (0,)), ((), ())),
                                   preferred_element_type=jnp.float32)
    off_excl_row = off_incl_row - pc_row
    ge8 = (er >= ec).astype(jnp.float32)
    off_incl_col = lax.dot_general(ge8, pc_col, (((1,), (0,)), ((), ())),
                                   preferred_element_type=jnp.float32)

    slot_val = excl + off_excl_row
    pos0 = jnp.sum(oh1 * slot_val, axis=1, keepdims=True)
    pos1 = jnp.sum(oh2 * slot_val, axis=1, keepdims=True)

    itile = lax.broadcasted_iota(jnp.int32, (E, 128), 1).astype(jnp.float32) * BLK
    ge_tiles = (itile >= off_incl_col).astype(jnp.float32)
    ones_e = jnp.full((1, E), 1.0, dtype=jnp.float32)
    te_row = lax.dot_general(ones_e, ge_tiles, (((1,), (0,)), ((), ())),
                             preferred_element_type=jnp.float32)
    te_row = jnp.minimum(te_row, float(E - 1))

    col8 = lax.broadcasted_iota(jnp.int32, (T, E), 1)
    meta = jnp.where(col8 == 0, pos0,
           jnp.where(col8 == 1, pos1,
           jnp.where(col8 == 2, p1,
           jnp.where(col8 == 3, p2, 0.0))))
    meta_ref[...] = meta
    te_ref[...] = jnp.broadcast_to(te_row, (8, 128))
    pw0_ref[...] = jnp.broadcast_to(p1, (T, 128))
    pw1_ref[...] = jnp.broadcast_to(p2, (T, 128))


def _router(xf, router_w, router_b):
    return pl.pallas_call(
        _router_body,
        out_shape=[jax.ShapeDtypeStruct((T, E), jnp.float32),
                   jax.ShapeDtypeStruct((8, 128), jnp.float32),
                   jax.ShapeDtypeStruct((T, 128), jnp.float32),
                   jax.ShapeDtypeStruct((T, 128), jnp.float32)],
    )(xf, router_w, router_b.reshape(1, E))


def _mlp_body(te_ref, gx_ref, upw_ref, upb_ref, dnw_ref, dnb_ref, ws_ref,
              out_ref):
    del te_ref
    gx = gx_ref[...]
    upw = upw_ref[0]
    h = lax.dot_general(gx, upw, (((1,), (1,)), ((), ())),
                        preferred_element_type=jnp.float32)
    h = h + upb_ref[0]
    a = h[:, :H]
    g = h[:, H:]
    hidden = a * (g * 0.5 * (1.0 + lax.erf(g * (2.0 ** -0.5))))
    dnw = dnw_ref[0]
    eo = lax.dot_general(hidden, dnw, (((1,), (1,)), ((), ())),
                         preferred_element_type=jnp.float32)
    out_ref[...] = (eo + dnb_ref[0]) * ws_ref[:, 0:1]


def _grouped_mlp(te, gx, up_w, up_b, down_w, down_b, ws):
    return pl.pallas_call(
        _mlp_body,
        grid_spec=pltpu.PrefetchScalarGridSpec(
            num_scalar_prefetch=1,
            grid=(N_TILES,),
            in_specs=[
                pl.BlockSpec((BLK, D), lambda i, te: (i, 0)),
                pl.BlockSpec((1, HU, D), lambda i, te: (te[i], 0, 0)),
                pl.BlockSpec((1, 1, HU), lambda i, te: (te[i], 0, 0)),
                pl.BlockSpec((1, D, H), lambda i, te: (te[i], 0, 0)),
                pl.BlockSpec((1, 1, D), lambda i, te: (te[i], 0, 0)),
                pl.BlockSpec((BLK, 128), lambda i, te: (i, 0)),
            ],
            out_specs=pl.BlockSpec((BLK, D), lambda i, te: (i, 0)),
        ),
        out_shape=jax.ShapeDtypeStruct((N_PAD, D), jnp.float32),
        compiler_params=pltpu.CompilerParams(
            dimension_semantics=("arbitrary",),
            vmem_limit_bytes=120 * 1024 * 1024,
        ),
    )(te, gx, up_w, up_b.reshape(E, 1, HU), down_w, down_b.reshape(E, 1, D),
      ws)


@functools.cache
def _dispatch_kernel():
    mesh = plsc.VectorSubcoreMesh(core_axis_name="c", subcore_axis_name="s")
    return pl.kernel(
        _dispatch_body,
        mesh=mesh,
        out_type=[jax.ShapeDtypeStruct((N_PAD, D), jnp.float32),
                  jax.ShapeDtypeStruct((N_PAD, 128), jnp.float32)],
        scratch_types=[
            pltpu.VMEM((TPW, D), jnp.float32),
            pltpu.VMEM((TPW, 128), jnp.float32),
            pltpu.VMEM((TPW,), jnp.int32),
            pltpu.VMEM((TPW,), jnp.int32),
            pltpu.SemaphoreType.DMA,
        ],
    )


def _dispatch_body(x_hbm, pos0_hbm, pos1_hbm, pw0_hbm, pw1_hbm,
                   gx_hbm, ws_hbm, rows_v, wrows_v, i0_v, i1_v, sem):
    wid = lax.axis_index("s") * 2 + lax.axis_index("c")
    base = wid * TPW
    pltpu.sync_copy(x_hbm.at[pl.ds(base, TPW)], rows_v)
    pltpu.sync_copy(pos0_hbm.at[pl.ds(base, TPW)], i0_v)
    pltpu.sync_copy(pos1_hbm.at[pl.ds(base, TPW)], i1_v)
    pltpu.async_copy(rows_v, gx_hbm.at[i0_v], sem).wait()
    pltpu.async_copy(rows_v, gx_hbm.at[i1_v], sem).wait()
    pltpu.sync_copy(pw0_hbm.at[pl.ds(base, TPW)], wrows_v)
    pltpu.async_copy(wrows_v, ws_hbm.at[i0_v], sem).wait()
    pltpu.sync_copy(pw1_hbm.at[pl.ds(base, TPW)], wrows_v)
    pltpu.async_copy(wrows_v, ws_hbm.at[i1_v], sem).wait()


@functools.cache
def _combine_kernel():
    mesh = plsc.VectorSubcoreMesh(core_axis_name="c", subcore_axis_name="s")
    return pl.kernel(
        _combine_body,
        mesh=mesh,
        out_type=jax.ShapeDtypeStruct((T, D), jnp.float32),
        scratch_types=[
            pltpu.VMEM((TPW, D), jnp.float32),
            pltpu.VMEM((TPW, D), jnp.float32),
            pltpu.VMEM((TPW,), jnp.int32),
            pltpu.VMEM((TPW,), jnp.int32),
            pltpu.SemaphoreType.DMA,
        ],
    )


def _combine_body(slot_hbm, pos0_hbm, pos1_hbm, out_hbm, r0, r1, i0, i1, sem):
    wid = lax.axis_index("s") * 2 + lax.axis_index("c")
    base = wid * TPW
    pltpu.sync_copy(pos0_hbm.at[pl.ds(base, TPW)], i0)
    pltpu.sync_copy(pos1_hbm.at[pl.ds(base, TPW)], i1)
    pltpu.async_copy(slot_hbm.at[i0], r0, sem).wait()
    pltpu.async_copy(slot_hbm.at[i1], r1, sem).wait()

    def row(i, carry):
        for j in range(D // LANES):
            sl = pl.ds(j * LANES, LANES)
            r0[i, sl] = r0[i, sl] + r1[i, sl]
        return carry

    lax.fori_loop(0, TPW, row, 0)
    pltpu.sync_copy(r0, out_hbm.at[pl.ds(base, TPW)])


def kernel(x, router_w, router_b, up_w, up_b, down_w, down_b):
    xf = x.reshape(T, D)
    meta, te8, pw0, pw1 = _router(xf, router_w, router_b)
    pos0 = meta[:, 0].astype(jnp.int32)
    pos1 = meta[:, 1].astype(jnp.int32)
    te = te8[0, :N_TILES].astype(jnp.int32)
    gx, ws = _dispatch_kernel()(xf, pos0, pos1, pw0, pw1)
    slot = _grouped_mlp(te, gx, up_w, up_b, down_w, down_b, ws)
    out = _combine_kernel()(slot, pos0, pos1)
    return out.reshape(1, T, D)


# speedup vs baseline: 4.7078x; 1.2071x over previous
import functools

import jax
import jax.numpy as jnp
from jax import lax
from jax.experimental import pallas as pl
from jax.experimental.pallas import tpu as pltpu
from jax.experimental.pallas import tpu_sc as plsc

T = 2048
D = 768
E = 8
H = 3072
HU = 6144
BLK = 256
N_PAD = 5888
N_TILES = N_PAD // BLK
NW = 32
TPW = T // NW
LANES = 16


def _router_body(x_ref, rw_ref, rb_ref, meta_ref, te_ref, pw0_ref, pw1_ref):
    x = x_ref[...]
    rw = rw_ref[...]
    logits = lax.dot_general(x, rw, (((1,), (1,)), ((), ())),
                             preferred_element_type=jnp.float32)
    logits = logits + rb_ref[...]

    eidx = lax.broadcasted_iota(jnp.int32, (T, E), 1)
    m1 = jnp.max(logits, axis=1, keepdims=True)
    is1 = logits == m1
    idx1 = jnp.min(jnp.where(is1, eidx, E), axis=1, keepdims=True)
    masked = jnp.where(eidx == idx1, -jnp.inf, logits)
    m2 = jnp.max(masked, axis=1, keepdims=True)
    idx2 = jnp.min(jnp.where(masked == m2, eidx, E), axis=1, keepdims=True)

    eb = jnp.exp(m2 - m1)
    p1 = 1.0 / (1.0 + eb)
    p2 = eb * p1

    oh1 = (eidx == idx1).astype(jnp.float32)
    oh2 = (eidx == idx2).astype(jnp.float32)
    m_sel = oh1 + oh2

    tr = lax.broadcasted_iota(jnp.int32, (T, T), 0)
    tc = lax.broadcasted_iota(jnp.int32, (T, T), 1)
    tri = (tr >= tc).astype(jnp.float32)
    cum = lax.dot_general(tri, m_sel, (((1,), (0,)), ((), ())),
                          preferred_element_type=jnp.float32)
    excl = cum - m_sel

    counts_row = cum[T - 1:T, :]
    ones_t = jnp.full((T, 1), 1.0, dtype=jnp.float32)
    counts_col = lax.dot_general(m_sel, ones_t, (((0,), (0,)), ((), ())),
                                 preferred_element_type=jnp.float32)

    inv_blk = 1.0 / BLK
    pc_row = jnp.floor((counts_row + (BLK - 1)) * inv_blk) * BLK
    pc_col = jnp.floor((counts_col + (BLK - 1)) * inv_blk) * BLK

    er = lax.broadcasted_iota(jnp.int32, (E, E), 0)
    ec = lax.broadcasted_iota(jnp.int32, (E, E), 1)
    le = (er <= ec).astype(jnp.float32)
    off_incl_row = lax.dot_general(pc_row, le, (((1,), (0,)), ((), ())),
                                   preferred_element_type=jnp.float32)
    off_excl_row = off_incl_row - pc_row
    ge8 = (er >= ec).astype(jnp.float32)
    off_incl_col = lax.dot_general(ge8, pc_col, (((1,), (0,)), ((), ())),
                                   preferred_element_type=jnp.float32)

    slot_val = excl + off_excl_row
    pos0 = jnp.sum(oh1 * slot_val, axis=1, keepdims=True)
    pos1 = jnp.sum(oh2 * slot_val, axis=1, keepdims=True)

    off_excl_col = off_incl_col - pc_col
    present = (pc_col > 0.0)
    itile = lax.broadcasted_iota(jnp.int32, (E, 128), 1).astype(jnp.float32) * BLK
    e_col = lax.broadcasted_iota(jnp.int32, (E, 128), 0).astype(jnp.float32)
    started = present & (off_excl_col <= itile)
    te_row = jnp.max(jnp.where(started, e_col, -1.0), axis=0, keepdims=True)
    nstart = jnp.sum(started.astype(jnp.float32), axis=0, keepdims=True)
    spm1 = nstart - 1.0
    sp_row = spm1 - 2.0 * jnp.floor(spm1 * 0.5)
    first_row = jnp.sum((present & (off_excl_col == itile)).astype(jnp.float32),
                        axis=0, keepdims=True)
    nxt_row = jnp.min(jnp.where(present & (off_excl_col > itile), e_col, 99.0),
                      axis=0, keepdims=True)

    col8 = lax.broadcasted_iota(jnp.int32, (T, E), 1)
    meta = jnp.where(col8 == 0, pos0,
           jnp.where(col8 == 1, pos1,
           jnp.where(col8 == 2, p1,
           jnp.where(col8 == 3, p2, 0.0))))
    meta_ref[...] = meta
    rowi = lax.broadcasted_iota(jnp.int32, (8, 128), 0)
    te_ref[...] = jnp.where(rowi == 0, te_row,
                  jnp.where(rowi == 1, nxt_row,
                  jnp.where(rowi == 2, sp_row, first_row)))
    pw0_ref[...] = jnp.broadcast_to(p1, (T, 128))
    pw1_ref[...] = jnp.broadcast_to(p2, (T, 128))


def _router(xf, router_w, router_b):
    return pl.pallas_call(
        _router_body,
        out_shape=[jax.ShapeDtypeStruct((T, E), jnp.float32),
                   jax.ShapeDtypeStruct((8, 128), jnp.float32),
                   jax.ShapeDtypeStruct((T, 128), jnp.float32),
                   jax.ShapeDtypeStruct((T, 128), jnp.float32)],
    )(xf, router_w, router_b.reshape(1, E))


def _mlp_body(info_ref, gx_ref, upw_hbm, upb_ref, dnw_hbm, dnb_ref, ws_ref,
              out_ref, upbuf, dnbuf, sem_up, sem_dn):
    i = pl.program_id(0)
    cur = info_ref[0, i]
    nxt = info_ref[1, i]
    sp = info_ref[2, i]
    fst = info_ref[3, i]

    @pl.when(i == 0)
    def _():
        pltpu.make_async_copy(upw_hbm.at[cur], upbuf.at[sp],
                              sem_up.at[sp]).start()
        pltpu.make_async_copy(dnw_hbm.at[cur], dnbuf.at[sp],
                              sem_dn.at[sp]).start()

    @pl.when(fst == 1)
    def _():
        pltpu.make_async_copy(upw_hbm.at[cur], upbuf.at[sp],
                              sem_up.at[sp]).wait()
        pltpu.make_async_copy(dnw_hbm.at[cur], dnbuf.at[sp],
                              sem_dn.at[sp]).wait()

    @pl.when((fst == 1) & (nxt < E))
    def _():
        pltpu.make_async_copy(upw_hbm.at[nxt], upbuf.at[1 - sp],
                              sem_up.at[1 - sp]).start()
        pltpu.make_async_copy(dnw_hbm.at[nxt], dnbuf.at[1 - sp],
                              sem_dn.at[1 - sp]).start()

    gx = gx_ref[...]
    h = lax.dot_general(gx, upbuf[sp], (((1,), (1,)), ((), ())),
                        preferred_element_type=jnp.float32)
    h = h + upb_ref[0]
    a = h[:, :H]
    g = h[:, H:]
    hidden = a * (g * 0.5 * (1.0 + lax.erf(g * (2.0 ** -0.5))))
    eo = lax.dot_general(hidden, dnbuf[sp], (((1,), (1,)), ((), ())),
                         preferred_element_type=jnp.float32)
    out_ref[...] = (eo + dnb_ref[0]) * ws_ref[:, 0:1]


def _grouped_mlp(info, gx, up_w, up_b, down_w, down_b, ws):
    return pl.pallas_call(
        _mlp_body,
        grid_spec=pltpu.PrefetchScalarGridSpec(
            num_scalar_prefetch=1,
            grid=(N_TILES,),
            in_specs=[
                pl.BlockSpec((BLK, D), lambda i, info: (i, 0)),
                pl.BlockSpec(memory_space=pl.ANY),
                pl.BlockSpec((1, 1, HU), lambda i, info: (info[0, i], 0, 0)),
                pl.BlockSpec(memory_space=pl.ANY),
                pl.BlockSpec((1, 1, D), lambda i, info: (info[0, i], 0, 0)),
                pl.BlockSpec((BLK, 128), lambda i, info: (i, 0)),
            ],
            out_specs=pl.BlockSpec((BLK, D), lambda i, info: (i, 0)),
            scratch_shapes=[
                pltpu.VMEM((2, HU, D), jnp.float32),
                pltpu.VMEM((2, D, H), jnp.float32),
                pltpu.SemaphoreType.DMA((2,)),
                pltpu.SemaphoreType.DMA((2,)),
            ],
        ),
        out_shape=jax.ShapeDtypeStruct((N_PAD, D), jnp.float32),
        compiler_params=pltpu.CompilerParams(
            dimension_semantics=("arbitrary",),
            vmem_limit_bytes=120 * 1024 * 1024,
        ),
    )(info, gx, up_w, up_b.reshape(E, 1, HU), down_w, down_b.reshape(E, 1, D),
      ws)


@functools.cache
def _dispatch_kernel():
    mesh = plsc.VectorSubcoreMesh(core_axis_name="c", subcore_axis_name="s")
    return pl.kernel(
        _dispatch_body,
        mesh=mesh,
        out_type=[jax.ShapeDtypeStruct((N_PAD, D), jnp.float32),
                  jax.ShapeDtypeStruct((N_PAD, 128), jnp.float32)],
        scratch_types=[
            pltpu.VMEM((TPW, D), jnp.float32),
            pltpu.VMEM((TPW, 128), jnp.float32),
            pltpu.VMEM((TPW,), jnp.int32),
            pltpu.VMEM((TPW,), jnp.int32),
            pltpu.SemaphoreType.DMA,
        ],
    )


def _dispatch_body(x_hbm, pos0_hbm, pos1_hbm, pw0_hbm, pw1_hbm,
                   gx_hbm, ws_hbm, rows_v, wrows_v, i0_v, i1_v, sem):
    wid = lax.axis_index("s") * 2 + lax.axis_index("c")
    base = wid * TPW
    pltpu.sync_copy(x_hbm.at[pl.ds(base, TPW)], rows_v)
    pltpu.sync_copy(pos0_hbm.at[pl.ds(base, TPW)], i0_v)
    pltpu.sync_copy(pos1_hbm.at[pl.ds(base, TPW)], i1_v)
    pltpu.async_copy(rows_v, gx_hbm.at[i0_v], sem).wait()
    pltpu.async_copy(rows_v, gx_hbm.at[i1_v], sem).wait()
    pltpu.sync_copy(pw0_hbm.at[pl.ds(base, TPW)], wrows_v)
    pltpu.async_copy(wrows_v, ws_hbm.at[i0_v], sem).wait()
    pltpu.sync_copy(pw1_hbm.at[pl.ds(base, TPW)], wrows_v)
    pltpu.async_copy(wrows_v, ws_hbm.at[i1_v], sem).wait()


@functools.cache
def _combine_kernel():
    mesh = plsc.VectorSubcoreMesh(core_axis_name="c", subcore_axis_name="s")
    return pl.kernel(
        _combine_body,
        mesh=mesh,
        out_type=jax.ShapeDtypeStruct((T, D), jnp.float32),
        scratch_types=[
            pltpu.VMEM((TPW, D), jnp.float32),
            pltpu.VMEM((TPW, D), jnp.float32),
            pltpu.VMEM((TPW,), jnp.int32),
            pltpu.VMEM((TPW,), jnp.int32),
            pltpu.SemaphoreType.DMA,
        ],
    )


def _combine_body(slot_hbm, pos0_hbm, pos1_hbm, out_hbm, r0, r1, i0, i1, sem):
    wid = lax.axis_index("s") * 2 + lax.axis_index("c")
    base = wid * TPW
    pltpu.sync_copy(pos0_hbm.at[pl.ds(base, TPW)], i0)
    pltpu.sync_copy(pos1_hbm.at[pl.ds(base, TPW)], i1)
    pltpu.async_copy(slot_hbm.at[i0], r0, sem).wait()
    pltpu.async_copy(slot_hbm.at[i1], r1, sem).wait()

    def row(i, carry):
        for j in range(D // LANES):
            sl = pl.ds(j * LANES, LANES)
            r0[i, sl] = r0[i, sl] + r1[i, sl]
        return carry

    lax.fori_loop(0, TPW, row, 0)
    pltpu.sync_copy(r0, out_hbm.at[pl.ds(base, TPW)])


def kernel(x, router_w, router_b, up_w, up_b, down_w, down_b):
    xf = x.reshape(T, D)
    meta, te8, pw0, pw1 = _router(xf, router_w, router_b)
    pos0 = meta[:, 0].astype(jnp.int32)
    pos1 = meta[:, 1].astype(jnp.int32)
    info = te8[:4, :N_TILES].astype(jnp.int32)
    gx, ws = _dispatch_kernel()(xf, pos0, pos1, pw0, pw1)
    slot = _grouped_mlp(info, gx, up_w, up_b, down_w, down_b, ws)
    out = _combine_kernel()(slot, pos0, pos1)
    return out.reshape(1, T, D)
